# baseline (speedup 1.0000x reference)
import functools

import jax
import jax.numpy as jnp
from jax import lax
from jax.experimental import pallas as pl
from jax.experimental.pallas import tpu as pltpu
from jax.experimental.pallas import tpu_sc as plsc

CUTOFF = 5.0
RBF_DIM = 16
ATOM_DIM = 128
HIDDEN = 128
LATENT = 64

NC, NS = 2, 16
NW = NC * NS
CH = 128
_COEFF = -0.5 / (CUTOFF / (RBF_DIM - 1)) ** 2


def _silu(x):
    return x / (1.0 + jnp.exp(-x))


def _sigmoid(x):
    return 1.0 / (1.0 + jnp.exp(-x))


def _node_tables(h_flat, z2, zpad, w1h, w1z, gs, gd):
    flat = h_flat.shape[0]
    nb = 2000
    grid = (flat // nb,)

    def body(h_ref, z_ref, zp_ref, w1h_ref, w1z_ref, gs_ref, gd_ref,
             noded_ref, nodes_ref):
        h = h_ref[...]
        zc = z_ref[...]
        oh = (zc == lax.broadcasted_iota(jnp.int32, (nb, 128), 1)
              ).astype(jnp.float32)
        wzp = jnp.dot(zp_ref[...], w1z_ref[...],
                      preferred_element_type=jnp.float32)
        node_m = (jnp.dot(h, w1h_ref[...], preferred_element_type=jnp.float32)
                  + jnp.dot(oh, wzp, preferred_element_type=jnp.float32))
        node_gd = jnp.dot(h, gd_ref[...], preferred_element_type=jnp.float32)
        noded_ref[...] = jnp.concatenate([node_m, node_gd],
                                         axis=1).astype(jnp.bfloat16)
        nodes_ref[...] = jnp.dot(h, gs_ref[...],
                                 preferred_element_type=jnp.float32
                                 ).astype(jnp.bfloat16)

    return pl.pallas_call(
        body,
        grid=grid,
        in_specs=[
            pl.BlockSpec((nb, 128), lambda i: (i, 0)),
            pl.BlockSpec((nb, 1), lambda i: (i, 0)),
            pl.BlockSpec((128, 32), lambda i: (0, 0)),
            pl.BlockSpec((128, 128), lambda i: (0, 0)),
            pl.BlockSpec((32, 128), lambda i: (0, 0)),
            pl.BlockSpec((128, 128), lambda i: (0, 0)),
            pl.BlockSpec((128, 128), lambda i: (0, 0)),
        ],
        out_specs=[
            pl.BlockSpec((nb, 256), lambda i: (i, 0)),
            pl.BlockSpec((nb, 128), lambda i: (i, 0)),
        ],
        out_shape=[
            jax.ShapeDtypeStruct((flat, 256), jnp.bfloat16),
            jax.ShapeDtypeStruct((flat, 128), jnp.bfloat16),
        ],
    )(h_flat, z2, zpad, w1h, w1z, gs, gd)


def _edge_gather(noded, nodes, dst_p, src_p, ep):
    ept = ep // NW
    nch = ept // CH
    mesh = plsc.VectorSubcoreMesh(core_axis_name="c", subcore_axis_name="s",
                                  num_cores=NC, num_subcores=NS)

    @functools.partial(
        pl.kernel,
        out_type=[
            jax.ShapeDtypeStruct((ep, 256), jnp.bfloat16),
            jax.ShapeDtypeStruct((ep, 128), jnp.bfloat16),
        ],
        mesh=mesh,
        compiler_params=pltpu.CompilerParams(use_tc_tiling_on_sc=False),
        scratch_types=[
            pltpu.VMEM((ept,), jnp.int32),
            pltpu.VMEM((ept,), jnp.int32),
            pltpu.VMEM((CH, 256), jnp.bfloat16),
            pltpu.VMEM((CH, 256), jnp.bfloat16),
            pltpu.VMEM((CH, 128), jnp.bfloat16),
            pltpu.VMEM((CH, 128), jnp.bfloat16),
            pltpu.SemaphoreType.DMA,
            pltpu.SemaphoreType.DMA,
            pltpu.SemaphoreType.DMA,
            pltpu.SemaphoreType.DMA,
            pltpu.SemaphoreType.DMA,
        ],
    )
    def k(noded_hbm, nodes_hbm, dst_hbm, src_hbm, outd_hbm, outs_hbm,
          idxd_v, idxs_v, bufd0, bufd1, bufs0, bufs1,
          gd0, gd1, gs0, gs1, wsem):
        wid = lax.axis_index("s") * NC + lax.axis_index("c")
        base = pl.multiple_of(wid * ept, CH)
        pltpu.sync_copy(dst_hbm.at[pl.ds(base, ept)], idxd_v)
        pltpu.sync_copy(src_hbm.at[pl.ds(base, ept)], idxs_v)

        def pair(i, carry):
            off = pl.multiple_of(i * (2 * CH), CH)
            dd0 = pltpu.async_copy(
                noded_hbm.at[idxd_v.at[pl.ds(off, CH)]], bufd0, gd0)
            ds0 = pltpu.async_copy(
                nodes_hbm.at[idxs_v.at[pl.ds(off, CH)]], bufs0, gs0)
            dd1 = pltpu.async_copy(
                noded_hbm.at[idxd_v.at[pl.ds(off + CH, CH)]], bufd1, gd1)
            ds1 = pltpu.async_copy(
                nodes_hbm.at[idxs_v.at[pl.ds(off + CH, CH)]], bufs1, gs1)
            dd0.wait()
            w0 = pltpu.async_copy(bufd0, outd_hbm.at[pl.ds(base + off, CH)],
                                  wsem)
            ds0.wait()
            w1 = pltpu.async_copy(bufs0, outs_hbm.at[pl.ds(base + off, CH)],
                                  wsem)
            dd1.wait()
            w2 = pltpu.async_copy(
                bufd1, outd_hbm.at[pl.ds(base + off + CH, CH)], wsem)
            ds1.wait()
            w3 = pltpu.async_copy(
                bufs1, outs_hbm.at[pl.ds(base + off + CH, CH)], wsem)
            w0.wait()
            w1.wait()
            w2.wait()
            w3.wait()
            return carry

        lax.fori_loop(0, nch // 2, pair, 0)

    return k(noded, nodes, dst_p, src_p)


def _edge_mlp(pred, pres, dist2, src2, dst2,
              w1r, w1s, b1, w2, b2, w3, b3, gr, gsr, gb1, gw2r, gb2r, ep):
    eb = 512
    grid = (ep // eb,)

    def body(pred_ref, pres_ref, d_ref, s_ref, t_ref, w1r_ref, w1s_ref, b1_ref,
             w2_ref, b2_ref, w3_ref, b3_ref, gr_ref, gsr_ref, gb1_ref,
             gw2_ref, gb2_ref, msg_ref):
        d = d_ref[...]
        cen = lax.broadcasted_iota(jnp.int32, (eb, RBF_DIM), 1).astype(
            jnp.float32) * (CUTOFF / (RBF_DIM - 1))
        rbf = jnp.exp(_COEFF * (d - cen) ** 2)
        isself = (s_ref[...] == t_ref[...]).astype(jnp.float32)
        pred_blk = pred_ref[...].astype(jnp.float32)
        pre_m = (pred_blk[:, :128]
                 + jnp.dot(rbf, w1r_ref[...], preferred_element_type=jnp.float32)
                 + isself * w1s_ref[...] + b1_ref[...])
        m1 = _silu(pre_m)
        m2 = _silu(jnp.dot(m1, w2_ref[...], preferred_element_type=jnp.float32)
                   + b2_ref[...])
        mv = jnp.dot(m2, w3_ref[...], preferred_element_type=jnp.float32) \
            + b3_ref[...]
        pre_g = (pres_ref[...].astype(jnp.float32) + pred_blk[:, 128:]
                 + jnp.dot(rbf, gr_ref[...], preferred_element_type=jnp.float32)
                 + isself * gsr_ref[...] + gb1_ref[...])
        g1 = _silu(pre_g)
        logit = jnp.sum(g1 * gw2_ref[...], axis=1, keepdims=True) + gb2_ref[...]
        gate = _sigmoid(logit)
        env = 0.5 * (jnp.cos(jnp.pi * d / CUTOFF) + 1.0) * (
            d < CUTOFF).astype(jnp.float32)
        msg_ref[...] = mv * (gate * env)

    full = lambda shape: pl.BlockSpec(shape, lambda i: tuple(0 for _ in shape))
    return pl.pallas_call(
        body,
        grid=grid,
        in_specs=[
            pl.BlockSpec((eb, 256), lambda i: (i, 0)),
            pl.BlockSpec((eb, 128), lambda i: (i, 0)),
            pl.BlockSpec((eb, 1), lambda i: (i, 0)),
            pl.BlockSpec((eb, 1), lambda i: (i, 0)),
            pl.BlockSpec((eb, 1), lambda i: (i, 0)),
            full((16, 128)), full((1, 128)), full((1, 128)),
            full((128, 128)), full((1, 128)), full((128, 64)), full((1, 64)),
            full((16, 128)), full((1, 128)), full((1, 128)),
            full((1, 128)), full((1, 1)),
        ],
        out_specs=pl.BlockSpec((eb, 64), lambda i: (i, 0)),
        out_shape=jax.ShapeDtypeStruct((ep, 64), jnp.float32),
    )(pred, pres, dist2, src2, dst2,
      w1r, w1s, b1, w2, b2, w3, b3, gr, gsr, gb1, gw2r, gb2r)


def _scatter_add(msg, src2d, half, ep):
    nchunks = ep // CH
    cht = nchunks // NS
    acc_rows = half + CH
    stripe = acc_rows // NS
    mesh = plsc.VectorSubcoreMesh(core_axis_name="c", subcore_axis_name="s",
                                  num_cores=NC, num_subcores=NS)

    @functools.partial(
        pl.kernel,
        out_type=jax.ShapeDtypeStruct((NC, acc_rows, LATENT), jnp.float32),
        mesh=mesh,
        compiler_params=pltpu.CompilerParams(use_tc_tiling_on_sc=False),
        scratch_types=[
            pltpu.VMEM((cht, CH), jnp.int32),
            pltpu.VMEM((CH, LATENT), jnp.float32),
            pltpu.VMEM((stripe, LATENT), jnp.float32),
            pltpu.VMEM_SHARED((acc_rows, LATENT), jnp.float32),
            pltpu.SemaphoreType.DMA,
        ],
    )
    def k(msg_hbm, src_hbm, part_hbm, idx_v, buf_v, obuf_v, acc_sh, sem):
        c = lax.axis_index("c")
        s = lax.axis_index("s")
        lo = c * half
        zv = jnp.zeros((16,), jnp.float32)

        def zrow(i, carry):
            for kk in range(LATENT // 16):
                obuf_v[i, pl.ds(kk * 16, 16)] = zv
            return carry

        lax.fori_loop(0, stripe, zrow, 0)
        pltpu.sync_copy(obuf_v, acc_sh.at[pl.ds(s * stripe, stripe)])
        pltpu.sync_copy(src_hbm.at[pl.ds(s * cht, cht)], idx_v)

        def remap(i, carry):
            for kk in range(CH // 16):
                g = idx_v[i, pl.ds(kk * 16, 16)]
                loc = g - lo
                ok = (loc >= 0) & (loc < half)
                idx_v[i, pl.ds(kk * 16, 16)] = jnp.where(ok, loc, half)
            return carry

        lax.fori_loop(0, cht, remap, 0)
        plsc.subcore_barrier()

        def chunk(j, carry):
            off = pl.multiple_of(s * cht * CH, CH) + j * CH
            pltpu.sync_copy(msg_hbm.at[pl.ds(off, CH)], buf_v)
            pltpu.sync_copy(buf_v, acc_sh.at[idx_v.at[j]], add=True)
            return carry

        lax.fori_loop(0, cht, chunk, 0)
        plsc.subcore_barrier()
        pltpu.sync_copy(acc_sh.at[pl.ds(s * stripe, stripe)], obuf_v)
        pltpu.sync_copy(obuf_v, part_hbm.at[c, pl.ds(s * stripe, stripe)])

    return k(msg, src2d)


def _output_mlp(part, efp, egw1, egb1, egw2, egb2, egw3, egb3,
                ow1, ob1, ow2, ob2, half, ne):
    nb = 640
    nbc = half // nb
    grid = (NC * nbc,)

    def body(p_ref, ef_ref, egw1_ref, egb1_ref, egw2_ref, egb2_ref,
             egw3_ref, egb3_ref, ow1_ref, ob1_ref, ow2_ref, ob2_ref, out_ref):
        p = p_ref[0]
        e1 = _silu(jnp.dot(ef_ref[...], egw1_ref[...],
                           preferred_element_type=jnp.float32) + egb1_ref[...])
        e2 = _silu(jnp.dot(e1, egw2_ref[...],
                           preferred_element_type=jnp.float32) + egb2_ref[...])
        egate = jnp.dot(e2, egw3_ref[...],
                        preferred_element_type=jnp.float32) + egb3_ref[...]
        for j in range(ne):
            x = p * egate[j:j + 1, :]
            o1 = _silu(jnp.dot(x, ow1_ref[...],
                               preferred_element_type=jnp.float32) + ob1_ref[...])
            o = jnp.dot(o1, ow2_ref[...],
                        preferred_element_type=jnp.float32) + ob2_ref[...]
            out_ref[:, j * LATENT:(j + 1) * LATENT] = o

    full = lambda shape: pl.BlockSpec(shape, lambda i: tuple(0 for _ in shape))
    return pl.pallas_call(
        body,
        grid=grid,
        in_specs=[
            pl.BlockSpec((1, nb, LATENT), lambda i: (i // nbc, i % nbc, 0)),
            full((8, 16)),
            full((16, 128)), full((1, 128)),
            full((128, 128)), full((1, 128)),
            full((128, 64)), full((1, 64)),
            full((64, 128)), full((1, 128)),
            full((128, 64)), full((1, 64)),
        ],
        out_specs=pl.BlockSpec((nb, ne * LATENT), lambda i: (i, 0)),
        out_shape=jax.ShapeDtypeStruct((NC * half, ne * LATENT), jnp.float32),
    )(part, efp, egw1, egb1, egw2, egb2, egw3, egb3, ow1, ob1, ow2, ob2)


def kernel(h, z, mask, e_feat, att_src, att_dst, att_dist, z_emb_W,
           msg_W1, msg_b1, msg_W2, msg_b2, msg_W3, msg_b3,
           gate_W1, gate_b1, gate_W2, gate_b2,
           eg_W1, eg_b1, eg_W2, eg_b2, eg_W3, eg_b3,
           out_W1, out_b1, out_W2, out_b2):
    bsz, n_atoms, h_dim = h.shape
    flat = bsz * n_atoms
    ne = e_feat.shape[0]
    e_att = att_src.shape[0]
    ep = ((e_att + NW * CH - 1) // (NW * CH)) * (NW * CH)
    pad = ep - e_att

    h_flat = h.reshape(flat, h_dim)
    z2 = z.reshape(flat, 1).astype(jnp.int32)
    zpad = jnp.pad(z_emb_W, ((0, 128 - z_emb_W.shape[0]), (0, 0)))

    w1h = msg_W1[0:128]
    w1z = msg_W1[128:160]
    w1s = msg_W1[160:161]
    w1r = msg_W1[161:177]
    gs = gate_W1[0:128]
    gd = gate_W1[128:256]
    gr = gate_W1[256:272]
    gsr = gate_W1[272:273]

    src_p = jnp.concatenate(
        [att_src.astype(jnp.int32), jnp.zeros((pad,), jnp.int32)])
    dst_p = jnp.concatenate(
        [att_dst.astype(jnp.int32), jnp.zeros((pad,), jnp.int32)])
    dist_p = jnp.concatenate(
        [att_dist.astype(jnp.float32), jnp.full((pad,), 2.0 * CUTOFF)])

    noded, nodes = _node_tables(h_flat, z2, zpad, w1h, w1z, gs, gd)
    pred, pres = _edge_gather(noded, nodes, dst_p, src_p, ep)
    msg = _edge_mlp(
        pred, pres, dist_p.reshape(ep, 1), src_p.reshape(ep, 1),
        dst_p.reshape(ep, 1),
        w1r, w1s, msg_b1.reshape(1, -1), msg_W2, msg_b2.reshape(1, -1),
        msg_W3, msg_b3.reshape(1, -1), gr, gsr, gate_b1.reshape(1, -1),
        gate_W2.reshape(1, -1), gate_b2.reshape(1, 1), ep)
    src2d = src_p.reshape(ep // CH, CH)
    half = ((flat + 2 * 8 * NS - 1) // (2 * 8 * NS)) * (8 * NS)
    part = _scatter_add(msg, src2d, half, ep)
    efp = jnp.pad(e_feat, ((0, 8 - ne), (0, 0)))
    out = _output_mlp(part, efp, eg_W1, eg_b1.reshape(1, -1), eg_W2,
                      eg_b2.reshape(1, -1), eg_W3, eg_b3.reshape(1, -1),
                      out_W1, out_b1.reshape(1, -1), out_W2,
                      out_b2.reshape(1, -1), half, ne)
    return out[:flat].reshape(bsz, n_atoms, ne, LATENT)


# speedup vs baseline: 1.1504x; 1.1504x over previous
import functools

import jax
import jax.numpy as jnp
from jax import lax
from jax.experimental import pallas as pl
from jax.experimental.pallas import tpu as pltpu
from jax.experimental.pallas import tpu_sc as plsc

CUTOFF = 5.0
RBF_DIM = 16
ATOM_DIM = 128
HIDDEN = 128
LATENT = 64

NC, NS = 2, 16
NW = NC * NS
CH = 128
_COEFF = -0.5 / (CUTOFF / (RBF_DIM - 1)) ** 2


def _silu(x):
    return x * (0.5 + 0.5 * jnp.tanh(0.5 * x))


def _sigmoid(x):
    return 0.5 + 0.5 * jnp.tanh(0.5 * x)


def _node_tables(h_flat, z2, zpad, w1h, w1z, gs, gd):
    flat = h_flat.shape[0]
    nb = 2000
    grid = (flat // nb,)

    def body(h_ref, z_ref, zp_ref, w1h_ref, w1z_ref, gs_ref, gd_ref,
             noded_ref, nodes_ref):
        h = h_ref[...]
        zc = z_ref[...]
        oh = (zc == lax.broadcasted_iota(jnp.int32, (nb, 128), 1)
              ).astype(jnp.float32)
        wzp = jnp.dot(zp_ref[...], w1z_ref[...],
                      preferred_element_type=jnp.float32)
        node_m = (jnp.dot(h, w1h_ref[...], preferred_element_type=jnp.float32)
                  + jnp.dot(oh, wzp, preferred_element_type=jnp.float32))
        node_gd = jnp.dot(h, gd_ref[...], preferred_element_type=jnp.float32)
        noded_ref[...] = jnp.concatenate([node_m, node_gd], axis=1)
        nodes_ref[...] = jnp.dot(h, gs_ref[...],
                                 preferred_element_type=jnp.float32)

    return pl.pallas_call(
        body,
        grid=grid,
        in_specs=[
            pl.BlockSpec((nb, 128), lambda i: (i, 0)),
            pl.BlockSpec((nb, 1), lambda i: (i, 0)),
            pl.BlockSpec((128, 32), lambda i: (0, 0)),
            pl.BlockSpec((128, 128), lambda i: (0, 0)),
            pl.BlockSpec((32, 128), lambda i: (0, 0)),
            pl.BlockSpec((128, 128), lambda i: (0, 0)),
            pl.BlockSpec((128, 128), lambda i: (0, 0)),
        ],
        out_specs=[
            pl.BlockSpec((nb, 256), lambda i: (i, 0)),
            pl.BlockSpec((nb, 128), lambda i: (i, 0)),
        ],
        out_shape=[
            jax.ShapeDtypeStruct((flat, 256), jnp.float32),
            jax.ShapeDtypeStruct((flat, 128), jnp.float32),
        ],
    )(h_flat, z2, zpad, w1h, w1z, gs, gd)


def _edge_gather(noded, nodes, dst_p, src_p, ep):
    ept = ep // NW
    nch = ept // CH
    mesh = plsc.VectorSubcoreMesh(core_axis_name="c", subcore_axis_name="s",
                                  num_cores=NC, num_subcores=NS)

    @functools.partial(
        pl.kernel,
        out_type=[
            jax.ShapeDtypeStruct((ep, 256), jnp.float32),
            jax.ShapeDtypeStruct((ep, 128), jnp.float32),
        ],
        mesh=mesh,
        compiler_params=pltpu.CompilerParams(use_tc_tiling_on_sc=False),
        scratch_types=[
            pltpu.VMEM((ept,), jnp.int32),
            pltpu.VMEM((ept,), jnp.int32),
            pltpu.VMEM((CH, 256), jnp.float32),
            pltpu.VMEM((CH, 256), jnp.float32),
            pltpu.VMEM((CH, 128), jnp.float32),
            pltpu.VMEM((CH, 128), jnp.float32),
            pltpu.SemaphoreType.DMA,
            pltpu.SemaphoreType.DMA,
            pltpu.SemaphoreType.DMA,
            pltpu.SemaphoreType.DMA,
            pltpu.SemaphoreType.DMA,
        ],
    )
    def k(noded_hbm, nodes_hbm, dst_hbm, src_hbm, outd_hbm, outs_hbm,
          idxd_v, idxs_v, bufd0, bufd1, bufs0, bufs1,
          gd0, gd1, gs0, gs1, wsem):
        wid = lax.axis_index("s") * NC + lax.axis_index("c")
        base = pl.multiple_of(wid * ept, CH)
        pltpu.sync_copy(dst_hbm.at[pl.ds(base, ept)], idxd_v)
        pltpu.sync_copy(src_hbm.at[pl.ds(base, ept)], idxs_v)

        def pair(i, carry):
            off = pl.multiple_of(i * (2 * CH), CH)
            dd0 = pltpu.async_copy(
                noded_hbm.at[idxd_v.at[pl.ds(off, CH)]], bufd0, gd0)
            ds0 = pltpu.async_copy(
                nodes_hbm.at[idxs_v.at[pl.ds(off, CH)]], bufs0, gs0)
            dd1 = pltpu.async_copy(
                noded_hbm.at[idxd_v.at[pl.ds(off + CH, CH)]], bufd1, gd1)
            ds1 = pltpu.async_copy(
                nodes_hbm.at[idxs_v.at[pl.ds(off + CH, CH)]], bufs1, gs1)
            dd0.wait()
            w0 = pltpu.async_copy(bufd0, outd_hbm.at[pl.ds(base + off, CH)],
                                  wsem)
            ds0.wait()
            w1 = pltpu.async_copy(bufs0, outs_hbm.at[pl.ds(base + off, CH)],
                                  wsem)
            dd1.wait()
            w2 = pltpu.async_copy(
                bufd1, outd_hbm.at[pl.ds(base + off + CH, CH)], wsem)
            ds1.wait()
            w3 = pltpu.async_copy(
                bufs1, outs_hbm.at[pl.ds(base + off + CH, CH)], wsem)
            w0.wait()
            w1.wait()
            w2.wait()
            w3.wait()
            return carry

        lax.fori_loop(0, nch // 2, pair, 0)

    return k(noded, nodes, dst_p, src_p)


def _edge_scalars(dist_2d, src_2d, dst_2d):
    rows = dist_2d.shape[0]

    def body(d_ref, s_ref, t_ref, env_ref, iss_ref):
        d = d_ref[...]
        env_ref[...] = 0.5 * (jnp.cos(jnp.pi * d / CUTOFF) + 1.0) * (
            d < CUTOFF).astype(jnp.float32)
        iss_ref[...] = (s_ref[...] == t_ref[...]).astype(jnp.float32)

    full = lambda shape: pl.BlockSpec(shape, lambda: tuple(0 for _ in shape))
    return pl.pallas_call(
        body,
        in_specs=[full((rows, CH))] * 3,
        out_specs=[full((rows, CH))] * 2,
        out_shape=[jax.ShapeDtypeStruct((rows, CH), jnp.float32)] * 2,
    )(dist_2d, src_2d, dst_2d)


def _edge_mlp(pred, pres, dist2, env2, iss2,
              w1r, w1s, b1, w2, b2, w3, b3, gr, gsr, gb1, gw2r, gb2r, ep):
    eb = 512
    grid = (ep // eb,)

    def body(pred_ref, pres_ref, d_ref, env_ref, iss_ref, w1r_ref, w1s_ref,
             b1_ref, w2_ref, b2_ref, w3_ref, b3_ref, gr_ref, gsr_ref, gb1_ref,
             gw2_ref, gb2_ref, msg_ref):
        d = d_ref[...]
        cen = lax.broadcasted_iota(jnp.int32, (eb, RBF_DIM), 1).astype(
            jnp.float32) * (CUTOFF / (RBF_DIM - 1))
        rbf = jnp.exp(_COEFF * (d - cen) ** 2)
        isself = iss_ref[...]
        pred_blk = pred_ref[...]
        pre_m = (pred_blk[:, :128]
                 + jnp.dot(rbf, w1r_ref[...], preferred_element_type=jnp.float32)
                 + isself * w1s_ref[...] + b1_ref[...])
        m1 = _silu(pre_m)
        m2 = _silu(jnp.dot(m1, w2_ref[...], preferred_element_type=jnp.float32)
                   + b2_ref[...])
        mv = jnp.dot(m2, w3_ref[...], preferred_element_type=jnp.float32) \
            + b3_ref[...]
        pre_g = (pres_ref[...] + pred_blk[:, 128:]
                 + jnp.dot(rbf, gr_ref[...], preferred_element_type=jnp.float32)
                 + isself * gsr_ref[...] + gb1_ref[...])
        g1 = _silu(pre_g)
        logit = jnp.sum(g1 * gw2_ref[...], axis=1, keepdims=True) + gb2_ref[...]
        gate = _sigmoid(logit)
        msg_ref[...] = mv * (gate * env_ref[...])

    full = lambda shape: pl.BlockSpec(shape, lambda i: tuple(0 for _ in shape))
    return pl.pallas_call(
        body,
        grid=grid,
        in_specs=[
            pl.BlockSpec((eb, 256), lambda i: (i, 0)),
            pl.BlockSpec((eb, 128), lambda i: (i, 0)),
            pl.BlockSpec((eb, 1), lambda i: (i, 0)),
            pl.BlockSpec((eb, 1), lambda i: (i, 0)),
            pl.BlockSpec((eb, 1), lambda i: (i, 0)),
            full((16, 128)), full((1, 128)), full((1, 128)),
            full((128, 128)), full((1, 128)), full((128, 64)), full((1, 64)),
            full((16, 128)), full((1, 128)), full((1, 128)),
            full((1, 128)), full((1, 1)),
        ],
        out_specs=pl.BlockSpec((eb, 64), lambda i: (i, 0)),
        out_shape=jax.ShapeDtypeStruct((ep, 64), jnp.float32),
    )(pred, pres, dist2, env2, iss2,
      w1r, w1s, b1, w2, b2, w3, b3, gr, gsr, gb1, gw2r, gb2r)


def _scatter_add(msg, src2d, half, ep):
    nchunks = ep // CH
    cht = nchunks // NS
    acc_rows = half + CH
    stripe = acc_rows // NS
    mesh = plsc.VectorSubcoreMesh(core_axis_name="c", subcore_axis_name="s",
                                  num_cores=NC, num_subcores=NS)

    @functools.partial(
        pl.kernel,
        out_type=jax.ShapeDtypeStruct((NC, acc_rows, LATENT), jnp.float32),
        mesh=mesh,
        compiler_params=pltpu.CompilerParams(use_tc_tiling_on_sc=False),
        scratch_types=[
            pltpu.VMEM((cht, CH), jnp.int32),
            pltpu.VMEM((CH, LATENT), jnp.float32),
            pltpu.VMEM((stripe, LATENT), jnp.float32),
            pltpu.VMEM_SHARED((acc_rows, LATENT), jnp.float32),
            pltpu.SemaphoreType.DMA,
        ],
    )
    def k(msg_hbm, src_hbm, part_hbm, idx_v, buf_v, obuf_v, acc_sh, sem):
        c = lax.axis_index("c")
        s = lax.axis_index("s")
        lo = c * half
        zv = jnp.zeros((16,), jnp.float32)

        def zrow(i, carry):
            for kk in range(LATENT // 16):
                obuf_v[i, pl.ds(kk * 16, 16)] = zv
            return carry

        lax.fori_loop(0, stripe, zrow, 0)
        pltpu.sync_copy(obuf_v, acc_sh.at[pl.ds(s * stripe, stripe)])
        pltpu.sync_copy(src_hbm.at[pl.ds(s * cht, cht)], idx_v)

        def remap(i, carry):
            for kk in range(CH // 16):
                g = idx_v[i, pl.ds(kk * 16, 16)]
                loc = g - lo
                ok = (loc >= 0) & (loc < half)
                idx_v[i, pl.ds(kk * 16, 16)] = jnp.where(ok, loc, half)
            return carry

        lax.fori_loop(0, cht, remap, 0)
        plsc.subcore_barrier()

        def chunk(j, carry):
            off = pl.multiple_of(s * cht * CH, CH) + j * CH
            pltpu.sync_copy(msg_hbm.at[pl.ds(off, CH)], buf_v)
            pltpu.sync_copy(buf_v, acc_sh.at[idx_v.at[j]], add=True)
            return carry

        lax.fori_loop(0, cht, chunk, 0)
        plsc.subcore_barrier()
        pltpu.sync_copy(acc_sh.at[pl.ds(s * stripe, stripe)], obuf_v)
        pltpu.sync_copy(obuf_v, part_hbm.at[c, pl.ds(s * stripe, stripe)])

    return k(msg, src2d)


def _output_mlp(part, efp, egw1, egb1, egw2, egb2, egw3, egb3,
                ow1, ob1, ow2, ob2, half, ne):
    nb = 640
    nbc = half // nb
    grid = (NC * nbc,)

    def body(p_ref, ef_ref, egw1_ref, egb1_ref, egw2_ref, egb2_ref,
             egw3_ref, egb3_ref, ow1_ref, ob1_ref, ow2_ref, ob2_ref, out_ref):
        p = p_ref[0]
        e1 = _silu(jnp.dot(ef_ref[...], egw1_ref[...],
                           preferred_element_type=jnp.float32) + egb1_ref[...])
        e2 = _silu(jnp.dot(e1, egw2_ref[...],
                           preferred_element_type=jnp.float32) + egb2_ref[...])
        egate = jnp.dot(e2, egw3_ref[...],
                        preferred_element_type=jnp.float32) + egb3_ref[...]
        for j in range(ne):
            x = p * egate[j:j + 1, :]
            o1 = _silu(jnp.dot(x, ow1_ref[...],
                               preferred_element_type=jnp.float32) + ob1_ref[...])
            o = jnp.dot(o1, ow2_ref[...],
                        preferred_element_type=jnp.float32) + ob2_ref[...]
            out_ref[:, j * LATENT:(j + 1) * LATENT] = o

    full = lambda shape: pl.BlockSpec(shape, lambda i: tuple(0 for _ in shape))
    return pl.pallas_call(
        body,
        grid=grid,
        in_specs=[
            pl.BlockSpec((1, nb, LATENT), lambda i: (i // nbc, i % nbc, 0)),
            full((8, 16)),
            full((16, 128)), full((1, 128)),
            full((128, 128)), full((1, 128)),
            full((128, 64)), full((1, 64)),
            full((64, 128)), full((1, 128)),
            full((128, 64)), full((1, 64)),
        ],
        out_specs=pl.BlockSpec((nb, ne * LATENT), lambda i: (i, 0)),
        out_shape=jax.ShapeDtypeStruct((NC * half, ne * LATENT), jnp.float32),
    )(part, efp, egw1, egb1, egw2, egb2, egw3, egb3, ow1, ob1, ow2, ob2)


def kernel(h, z, mask, e_feat, att_src, att_dst, att_dist, z_emb_W,
           msg_W1, msg_b1, msg_W2, msg_b2, msg_W3, msg_b3,
           gate_W1, gate_b1, gate_W2, gate_b2,
           eg_W1, eg_b1, eg_W2, eg_b2, eg_W3, eg_b3,
           out_W1, out_b1, out_W2, out_b2):
    bsz, n_atoms, h_dim = h.shape
    flat = bsz * n_atoms
    ne = e_feat.shape[0]
    e_att = att_src.shape[0]
    ep = ((e_att + NW * CH - 1) // (NW * CH)) * (NW * CH)
    pad = ep - e_att

    h_flat = h.reshape(flat, h_dim)
    z2 = z.reshape(flat, 1).astype(jnp.int32)
    zpad = jnp.pad(z_emb_W, ((0, 128 - z_emb_W.shape[0]), (0, 0)))

    w1h = msg_W1[0:128]
    w1z = msg_W1[128:160]
    w1s = msg_W1[160:161]
    w1r = msg_W1[161:177]
    gs = gate_W1[0:128]
    gd = gate_W1[128:256]
    gr = gate_W1[256:272]
    gsr = gate_W1[272:273]

    src_p = jnp.concatenate(
        [att_src.astype(jnp.int32), jnp.zeros((pad,), jnp.int32)])
    dst_p = jnp.concatenate(
        [att_dst.astype(jnp.int32), jnp.zeros((pad,), jnp.int32)])
    dist_p = jnp.concatenate(
        [att_dist.astype(jnp.float32), jnp.full((pad,), 2.0 * CUTOFF)])

    noded, nodes = _node_tables(h_flat, z2, zpad, w1h, w1z, gs, gd)
    pred, pres = _edge_gather(noded, nodes, dst_p, src_p, ep)
    env_2d, iss_2d = _edge_scalars(dist_p.reshape(ep // CH, CH),
                                   src_p.reshape(ep // CH, CH),
                                   dst_p.reshape(ep // CH, CH))
    msg = _edge_mlp(
        pred, pres, dist_p.reshape(ep, 1), env_2d.reshape(ep, 1),
        iss_2d.reshape(ep, 1),
        w1r, w1s, msg_b1.reshape(1, -1), msg_W2, msg_b2.reshape(1, -1),
        msg_W3, msg_b3.reshape(1, -1), gr, gsr, gate_b1.reshape(1, -1),
        gate_W2.reshape(1, -1), gate_b2.reshape(1, 1), ep)
    src2d = src_p.reshape(ep // CH, CH)
    half = ((flat + 2 * 8 * NS - 1) // (2 * 8 * NS)) * (8 * NS)
    part = _scatter_add(msg, src2d, half, ep)
    efp = jnp.pad(e_feat, ((0, 8 - ne), (0, 0)))
    out = _output_mlp(part, efp, eg_W1, eg_b1.reshape(1, -1), eg_W2,
                      eg_b2.reshape(1, -1), eg_W3, eg_b3.reshape(1, -1),
                      out_W1, out_b1.reshape(1, -1), out_W2,
                      out_b2.reshape(1, -1), half, ne)
    return out[:flat].reshape(bsz, n_atoms, ne, LATENT)


# speedup vs baseline: 1.4148x; 1.2299x over previous
import functools

import jax
import jax.numpy as jnp
from jax import lax
from jax.experimental import pallas as pl
from jax.experimental.pallas import tpu as pltpu
from jax.experimental.pallas import tpu_sc as plsc

CUTOFF = 5.0
RBF_DIM = 16
ATOM_DIM = 128
HIDDEN = 128
LATENT = 64

NC, NS = 2, 16
NW = NC * NS
CH = 128
_COEFF = -0.5 / (CUTOFF / (RBF_DIM - 1)) ** 2


def _silu(x):
    return x * (0.5 + 0.5 * jnp.tanh(0.5 * x))


def _sigmoid(x):
    return 0.5 + 0.5 * jnp.tanh(0.5 * x)


def _node_tables(h_flat, z2, zpad, w1h, w1z, gs, gd):
    flat = h_flat.shape[0]
    nb = 2000
    grid = (flat // nb,)

    def body(h_ref, z_ref, zp_ref, w1h_ref, w1z_ref, gs_ref, gd_ref,
             noded_ref, nodes_ref):
        h = h_ref[...]
        zc = z_ref[...]
        oh = (zc == lax.broadcasted_iota(jnp.int32, (nb, 128), 1)
              ).astype(jnp.float32)
        wzp = jnp.dot(zp_ref[...], w1z_ref[...],
                      preferred_element_type=jnp.float32)
        node_m = (jnp.dot(h, w1h_ref[...], preferred_element_type=jnp.float32)
                  + jnp.dot(oh, wzp, preferred_element_type=jnp.float32))
        node_gd = jnp.dot(h, gd_ref[...], preferred_element_type=jnp.float32)
        noded_ref[...] = jnp.concatenate([node_m, node_gd], axis=1)
        nodes_ref[...] = jnp.dot(h, gs_ref[...],
                                 preferred_element_type=jnp.float32)

    return pl.pallas_call(
        body,
        grid=grid,
        in_specs=[
            pl.BlockSpec((nb, 128), lambda i: (i, 0)),
            pl.BlockSpec((nb, 1), lambda i: (i, 0)),
            pl.BlockSpec((128, 32), lambda i: (0, 0)),
            pl.BlockSpec((128, 128), lambda i: (0, 0)),
            pl.BlockSpec((32, 128), lambda i: (0, 0)),
            pl.BlockSpec((128, 128), lambda i: (0, 0)),
            pl.BlockSpec((128, 128), lambda i: (0, 0)),
        ],
        out_specs=[
            pl.BlockSpec((nb, 256), lambda i: (i, 0)),
            pl.BlockSpec((nb, 128), lambda i: (i, 0)),
        ],
        out_shape=[
            jax.ShapeDtypeStruct((flat, 256), jnp.float32),
            jax.ShapeDtypeStruct((flat, 128), jnp.float32),
        ],
    )(h_flat, z2, zpad, w1h, w1z, gs, gd)


def _edge_gather(noded, nodes, dst_p, src_p, ep):
    ept = ep // NW
    nch = ept // CH
    mesh = plsc.VectorSubcoreMesh(core_axis_name="c", subcore_axis_name="s",
                                  num_cores=NC, num_subcores=NS)

    @functools.partial(
        pl.kernel,
        out_type=[
            jax.ShapeDtypeStruct((ep, 128), jnp.float32),
            jax.ShapeDtypeStruct((ep, 128), jnp.float32),
            jax.ShapeDtypeStruct((ep, 128), jnp.float32),
        ],
        mesh=mesh,
        compiler_params=pltpu.CompilerParams(use_tc_tiling_on_sc=False),
        scratch_types=[
            pltpu.VMEM((ept,), jnp.int32),
            pltpu.VMEM((ept,), jnp.int32),
            pltpu.VMEM((CH, 256), jnp.float32),
            pltpu.VMEM((CH, 256), jnp.float32),
            pltpu.VMEM((CH, 128), jnp.float32),
            pltpu.VMEM((CH, 128), jnp.float32),
            pltpu.SemaphoreType.DMA,
            pltpu.SemaphoreType.DMA,
            pltpu.SemaphoreType.DMA,
            pltpu.SemaphoreType.DMA,
            pltpu.SemaphoreType.DMA,
        ],
    )
    def k(noded_hbm, nodes_hbm, dst_hbm, src_hbm, outm_hbm, outg_hbm,
          outs_hbm, idxd_v, idxs_v, bufd0, bufd1, bufs0, bufs1,
          gd0, gd1, gs0, gs1, wsem):
        wid = lax.axis_index("s") * NC + lax.axis_index("c")
        base = pl.multiple_of(wid * ept, CH)
        pltpu.sync_copy(dst_hbm.at[pl.ds(base, ept)], idxd_v)
        pltpu.sync_copy(src_hbm.at[pl.ds(base, ept)], idxs_v)

        def pair(i, carry):
            off = pl.multiple_of(i * (2 * CH), CH)
            dd0 = pltpu.async_copy(
                noded_hbm.at[idxd_v.at[pl.ds(off, CH)]], bufd0, gd0)
            ds0 = pltpu.async_copy(
                nodes_hbm.at[idxs_v.at[pl.ds(off, CH)]], bufs0, gs0)
            dd1 = pltpu.async_copy(
                noded_hbm.at[idxd_v.at[pl.ds(off + CH, CH)]], bufd1, gd1)
            ds1 = pltpu.async_copy(
                nodes_hbm.at[idxs_v.at[pl.ds(off + CH, CH)]], bufs1, gs1)
            dd0.wait()
            w0 = pltpu.async_copy(
                bufd0.at[:, pl.ds(0, 128)],
                outm_hbm.at[pl.ds(base + off, CH)], wsem)
            w0b = pltpu.async_copy(
                bufd0.at[:, pl.ds(128, 128)],
                outg_hbm.at[pl.ds(base + off, CH)], wsem)
            ds0.wait()
            w1 = pltpu.async_copy(bufs0, outs_hbm.at[pl.ds(base + off, CH)],
                                  wsem)
            dd1.wait()
            w2 = pltpu.async_copy(
                bufd1.at[:, pl.ds(0, 128)],
                outm_hbm.at[pl.ds(base + off + CH, CH)], wsem)
            w2b = pltpu.async_copy(
                bufd1.at[:, pl.ds(128, 128)],
                outg_hbm.at[pl.ds(base + off + CH, CH)], wsem)
            ds1.wait()
            w3 = pltpu.async_copy(
                bufs1, outs_hbm.at[pl.ds(base + off + CH, CH)], wsem)
            w0.wait()
            w0b.wait()
            w1.wait()
            w2.wait()
            w2b.wait()
            w3.wait()
            return carry

        lax.fori_loop(0, nch // 2, pair, 0)

    return k(noded, nodes, dst_p, src_p)


def _edge_scalars(dist_2d, src_2d, dst_2d):
    rows = dist_2d.shape[0]

    def body(d_ref, s_ref, t_ref, env_ref, iss_ref):
        d = d_ref[...]
        env_ref[...] = 0.5 * (jnp.cos(jnp.pi * d / CUTOFF) + 1.0) * (
            d < CUTOFF).astype(jnp.float32)
        iss_ref[...] = (s_ref[...] == t_ref[...]).astype(jnp.float32)

    full = lambda shape: pl.BlockSpec(shape, lambda: tuple(0 for _ in shape))
    return pl.pallas_call(
        body,
        in_specs=[full((rows, CH))] * 3,
        out_specs=[full((rows, CH))] * 2,
        out_shape=[jax.ShapeDtypeStruct((rows, CH), jnp.float32)] * 2,
    )(dist_2d, src_2d, dst_2d)


def _edge_mlp(predm, predg, pres, dist2, env2, iss2,
              w1r, w1s, b1, w2, b2, w3, b3, gr, gsr, gb1, gw2r, gb2r, ep):
    eb = 1024
    grid = (ep // eb,)

    def body(predm_ref, predg_ref, pres_ref, d_ref, env_ref, iss_ref,
             w1r_ref, w1s_ref,
             b1_ref, w2_ref, b2_ref, w3_ref, b3_ref, gr_ref, gsr_ref, gb1_ref,
             gw2_ref, gb2_ref, msg_ref):
        d = d_ref[...]
        cen = lax.broadcasted_iota(jnp.int32, (eb, RBF_DIM), 1).astype(
            jnp.float32) * (CUTOFF / (RBF_DIM - 1))
        rbf = jnp.exp(_COEFF * (d - cen) ** 2)
        isself = iss_ref[...]
        pre_m = (predm_ref[...]
                 + jnp.dot(rbf, w1r_ref[...], preferred_element_type=jnp.float32)
                 + isself * w1s_ref[...] + b1_ref[...])
        m1 = _silu(pre_m)
        m2 = _silu(jnp.dot(m1, w2_ref[...], preferred_element_type=jnp.float32)
                   + b2_ref[...])
        mv = jnp.dot(m2, w3_ref[...], preferred_element_type=jnp.float32) \
            + b3_ref[...]
        pre_g = (pres_ref[...] + predg_ref[...]
                 + jnp.dot(rbf, gr_ref[...], preferred_element_type=jnp.float32)
                 + isself * gsr_ref[...] + gb1_ref[...])
        g1 = _silu(pre_g)
        logit = jnp.sum(g1 * gw2_ref[...], axis=1, keepdims=True) + gb2_ref[...]
        gate = _sigmoid(logit)
        msg_ref[...] = mv * (gate * env_ref[...])

    full = lambda shape: pl.BlockSpec(shape, lambda i: tuple(0 for _ in shape))
    return pl.pallas_call(
        body,
        grid=grid,
        in_specs=[
            pl.BlockSpec((eb, 128), lambda i: (i, 0)),
            pl.BlockSpec((eb, 128), lambda i: (i, 0)),
            pl.BlockSpec((eb, 128), lambda i: (i, 0)),
            pl.BlockSpec((eb, 1), lambda i: (i, 0)),
            pl.BlockSpec((eb, 1), lambda i: (i, 0)),
            pl.BlockSpec((eb, 1), lambda i: (i, 0)),
            full((16, 128)), full((1, 128)), full((1, 128)),
            full((128, 128)), full((1, 128)), full((128, 64)), full((1, 64)),
            full((16, 128)), full((1, 128)), full((1, 128)),
            full((1, 128)), full((1, 1)),
        ],
        out_specs=pl.BlockSpec((eb, 64), lambda i: (i, 0)),
        out_shape=jax.ShapeDtypeStruct((ep, 64), jnp.float32),
    )(predm, predg, pres, dist2, env2, iss2,
      w1r, w1s, b1, w2, b2, w3, b3, gr, gsr, gb1, gw2r, gb2r)


def _scatter_add(msg, src2d, half, ep):
    nchunks = ep // CH
    cht = nchunks // NS
    acc_rows = half + CH
    stripe = acc_rows // NS
    mesh = plsc.VectorSubcoreMesh(core_axis_name="c", subcore_axis_name="s",
                                  num_cores=NC, num_subcores=NS)

    @functools.partial(
        pl.kernel,
        out_type=jax.ShapeDtypeStruct((NC, acc_rows, LATENT), jnp.float32),
        mesh=mesh,
        compiler_params=pltpu.CompilerParams(use_tc_tiling_on_sc=False),
        scratch_types=[
            pltpu.VMEM((cht, CH), jnp.int32),
            pltpu.VMEM((CH, LATENT), jnp.float32),
            pltpu.VMEM((stripe, LATENT), jnp.float32),
            pltpu.VMEM_SHARED((acc_rows, LATENT), jnp.float32),
            pltpu.SemaphoreType.DMA,
        ],
    )
    def k(msg_hbm, src_hbm, part_hbm, idx_v, buf_v, obuf_v, acc_sh, sem):
        c = lax.axis_index("c")
        s = lax.axis_index("s")
        lo = c * half
        zv = jnp.zeros((16,), jnp.float32)

        def zrow(i, carry):
            for kk in range(LATENT // 16):
                obuf_v[i, pl.ds(kk * 16, 16)] = zv
            return carry

        lax.fori_loop(0, stripe, zrow, 0)
        pltpu.sync_copy(obuf_v, acc_sh.at[pl.ds(s * stripe, stripe)])
        pltpu.sync_copy(src_hbm.at[pl.ds(s * cht, cht)], idx_v)

        def remap(i, carry):
            for kk in range(CH // 16):
                g = idx_v[i, pl.ds(kk * 16, 16)]
                loc = g - lo
                ok = (loc >= 0) & (loc < half)
                idx_v[i, pl.ds(kk * 16, 16)] = jnp.where(ok, loc, half)
            return carry

        lax.fori_loop(0, cht, remap, 0)
        plsc.subcore_barrier()

        def chunk(j, carry):
            off = pl.multiple_of(s * cht * CH, CH) + j * CH
            pltpu.sync_copy(msg_hbm.at[pl.ds(off, CH)], buf_v)
            pltpu.sync_copy(buf_v, acc_sh.at[idx_v.at[j]], add=True)
            return carry

        lax.fori_loop(0, cht, chunk, 0)
        plsc.subcore_barrier()
        pltpu.sync_copy(acc_sh.at[pl.ds(s * stripe, stripe)], obuf_v)
        pltpu.sync_copy(obuf_v, part_hbm.at[c, pl.ds(s * stripe, stripe)])

    return k(msg, src2d)


def _output_mlp(part, efp, egw1, egb1, egw2, egb2, egw3, egb3,
                ow1, ob1, ow2, ob2, half, ne):
    nb = 640
    nbc = half // nb
    grid = (NC * nbc,)

    def body(p_ref, ef_ref, egw1_ref, egb1_ref, egw2_ref, egb2_ref,
             egw3_ref, egb3_ref, ow1_ref, ob1_ref, ow2_ref, ob2_ref, out_ref):
        p = p_ref[0]
        e1 = _silu(jnp.dot(ef_ref[...], egw1_ref[...],
                           preferred_element_type=jnp.float32) + egb1_ref[...])
        e2 = _silu(jnp.dot(e1, egw2_ref[...],
                           preferred_element_type=jnp.float32) + egb2_ref[...])
        egate = jnp.dot(e2, egw3_ref[...],
                        preferred_element_type=jnp.float32) + egb3_ref[...]
        for j in range(ne):
            x = p * egate[j:j + 1, :]
            o1 = _silu(jnp.dot(x, ow1_ref[...],
                               preferred_element_type=jnp.float32) + ob1_ref[...])
            o = jnp.dot(o1, ow2_ref[...],
                        preferred_element_type=jnp.float32) + ob2_ref[...]
            out_ref[:, j * LATENT:(j + 1) * LATENT] = o

    full = lambda shape: pl.BlockSpec(shape, lambda i: tuple(0 for _ in shape))
    return pl.pallas_call(
        body,
        grid=grid,
        in_specs=[
            pl.BlockSpec((1, nb, LATENT), lambda i: (i // nbc, i % nbc, 0)),
            full((8, 16)),
            full((16, 128)), full((1, 128)),
            full((128, 128)), full((1, 128)),
            full((128, 64)), full((1, 64)),
            full((64, 128)), full((1, 128)),
            full((128, 64)), full((1, 64)),
        ],
        out_specs=pl.BlockSpec((nb, ne * LATENT), lambda i: (i, 0)),
        out_shape=jax.ShapeDtypeStruct((NC * half, ne * LATENT), jnp.float32),
    )(part, efp, egw1, egb1, egw2, egb2, egw3, egb3, ow1, ob1, ow2, ob2)


def kernel(h, z, mask, e_feat, att_src, att_dst, att_dist, z_emb_W,
           msg_W1, msg_b1, msg_W2, msg_b2, msg_W3, msg_b3,
           gate_W1, gate_b1, gate_W2, gate_b2,
           eg_W1, eg_b1, eg_W2, eg_b2, eg_W3, eg_b3,
           out_W1, out_b1, out_W2, out_b2):
    bsz, n_atoms, h_dim = h.shape
    flat = bsz * n_atoms
    ne = e_feat.shape[0]
    e_att = att_src.shape[0]
    ep = ((e_att + NW * CH - 1) // (NW * CH)) * (NW * CH)
    pad = ep - e_att

    h_flat = h.reshape(flat, h_dim)
    z2 = z.reshape(flat, 1).astype(jnp.int32)
    zpad = jnp.pad(z_emb_W, ((0, 128 - z_emb_W.shape[0]), (0, 0)))

    w1h = msg_W1[0:128]
    w1z = msg_W1[128:160]
    w1s = msg_W1[160:161]
    w1r = msg_W1[161:177]
    gs = gate_W1[0:128]
    gd = gate_W1[128:256]
    gr = gate_W1[256:272]
    gsr = gate_W1[272:273]

    src_p = jnp.concatenate(
        [att_src.astype(jnp.int32), jnp.zeros((pad,), jnp.int32)])
    dst_p = jnp.concatenate(
        [att_dst.astype(jnp.int32), jnp.zeros((pad,), jnp.int32)])
    dist_p = jnp.concatenate(
        [att_dist.astype(jnp.float32), jnp.full((pad,), 2.0 * CUTOFF)])

    noded, nodes = _node_tables(h_flat, z2, zpad, w1h, w1z, gs, gd)
    predm, predg, pres = _edge_gather(noded, nodes, dst_p, src_p, ep)
    env_2d, iss_2d = _edge_scalars(dist_p.reshape(ep // CH, CH),
                                   src_p.reshape(ep // CH, CH),
                                   dst_p.reshape(ep // CH, CH))
    msg = _edge_mlp(
        predm, predg, pres, dist_p.reshape(ep, 1), env_2d.reshape(ep, 1),
        iss_2d.reshape(ep, 1),
        w1r, w1s, msg_b1.reshape(1, -1), msg_W2, msg_b2.reshape(1, -1),
        msg_W3, msg_b3.reshape(1, -1), gr, gsr, gate_b1.reshape(1, -1),
        gate_W2.reshape(1, -1), gate_b2.reshape(1, 1), ep)
    src2d = src_p.reshape(ep // CH, CH)
    half = ((flat + 2 * 8 * NS - 1) // (2 * 8 * NS)) * (8 * NS)
    part = _scatter_add(msg, src2d, half, ep)
    efp = jnp.pad(e_feat, ((0, 8 - ne), (0, 0)))
    out = _output_mlp(part, efp, eg_W1, eg_b1.reshape(1, -1), eg_W2,
                      eg_b2.reshape(1, -1), eg_W3, eg_b3.reshape(1, -1),
                      out_W1, out_b1.reshape(1, -1), out_W2,
                      out_b2.reshape(1, -1), half, ne)
    return out[:flat].reshape(bsz, n_atoms, ne, LATENT)


# speedup vs baseline: 1.4965x; 1.0577x over previous
import functools

import jax
import jax.numpy as jnp
from jax import lax
from jax.experimental import pallas as pl
from jax.experimental.pallas import tpu as pltpu
from jax.experimental.pallas import tpu_sc as plsc

CUTOFF = 5.0
RBF_DIM = 16
ATOM_DIM = 128
HIDDEN = 128
LATENT = 64

NC, NS = 2, 16
NW = NC * NS
CH = 128
_COEFF = -0.5 / (CUTOFF / (RBF_DIM - 1)) ** 2


def _silu(x):
    return x * (0.5 + 0.5 * jnp.tanh(0.5 * x))


def _sigmoid(x):
    return 0.5 + 0.5 * jnp.tanh(0.5 * x)


def _node_tables(h_flat, z2, zpad, w1h, w1z, gs, gd):
    flat = h_flat.shape[0]
    nb = 2000
    grid = (flat // nb,)

    def body(h_ref, z_ref, zp_ref, w1h_ref, w1z_ref, gs_ref, gd_ref,
             noded_ref, nodes_ref):
        h = h_ref[...]
        zc = z_ref[...]
        oh = (zc == lax.broadcasted_iota(jnp.int32, (nb, 128), 1)
              ).astype(jnp.float32)
        wzp = jnp.dot(zp_ref[...], w1z_ref[...],
                      preferred_element_type=jnp.float32)
        node_m = (jnp.dot(h, w1h_ref[...], preferred_element_type=jnp.float32)
                  + jnp.dot(oh, wzp, preferred_element_type=jnp.float32))
        node_gd = jnp.dot(h, gd_ref[...], preferred_element_type=jnp.float32)
        noded_ref[...] = jnp.concatenate([node_m, node_gd], axis=1)
        nodes_ref[...] = jnp.dot(h, gs_ref[...],
                                 preferred_element_type=jnp.float32)

    return pl.pallas_call(
        body,
        grid=grid,
        in_specs=[
            pl.BlockSpec((nb, 128), lambda i: (i, 0)),
            pl.BlockSpec((nb, 1), lambda i: (i, 0)),
            pl.BlockSpec((128, 32), lambda i: (0, 0)),
            pl.BlockSpec((128, 128), lambda i: (0, 0)),
            pl.BlockSpec((32, 128), lambda i: (0, 0)),
            pl.BlockSpec((128, 128), lambda i: (0, 0)),
            pl.BlockSpec((128, 128), lambda i: (0, 0)),
        ],
        out_specs=[
            pl.BlockSpec((nb, 256), lambda i: (i, 0)),
            pl.BlockSpec((nb, 128), lambda i: (i, 0)),
        ],
        out_shape=[
            jax.ShapeDtypeStruct((flat, 256), jnp.float32),
            jax.ShapeDtypeStruct((flat, 128), jnp.float32),
        ],
    )(h_flat, z2, zpad, w1h, w1z, gs, gd)


def _edge_gather(noded, nodes, dst_p, src_p, ep):
    ept = ep // NW
    nch = ept // CH
    mesh = plsc.VectorSubcoreMesh(core_axis_name="c", subcore_axis_name="s",
                                  num_cores=NC, num_subcores=NS)

    @functools.partial(
        pl.kernel,
        out_type=[
            jax.ShapeDtypeStruct((ep, 128), jnp.float32),
            jax.ShapeDtypeStruct((ep, 128), jnp.float32),
            jax.ShapeDtypeStruct((ep, 128), jnp.float32),
        ],
        mesh=mesh,
        compiler_params=pltpu.CompilerParams(use_tc_tiling_on_sc=False),
        scratch_types=[
            pltpu.VMEM((ept,), jnp.int32),
            pltpu.VMEM((ept,), jnp.int32),
            pltpu.VMEM((CH, 256), jnp.float32),
            pltpu.VMEM((CH, 256), jnp.float32),
            pltpu.VMEM((CH, 128), jnp.float32),
            pltpu.VMEM((CH, 128), jnp.float32),
            pltpu.SemaphoreType.DMA,
            pltpu.SemaphoreType.DMA,
            pltpu.SemaphoreType.DMA,
            pltpu.SemaphoreType.DMA,
            pltpu.SemaphoreType.DMA,
        ],
    )
    def k(noded_hbm, nodes_hbm, dst_hbm, src_hbm, outm_hbm, outg_hbm,
          outs_hbm, idxd_v, idxs_v, bufd0, bufd1, bufs0, bufs1,
          gd0, gd1, gs0, gs1, wsem):
        wid = lax.axis_index("s") * NC + lax.axis_index("c")
        base = pl.multiple_of(wid * ept, CH)
        pltpu.sync_copy(dst_hbm.at[pl.ds(base, ept)], idxd_v)
        pltpu.sync_copy(src_hbm.at[pl.ds(base, ept)], idxs_v)

        def pair(i, carry):
            off = pl.multiple_of(i * (2 * CH), CH)
            dd0 = pltpu.async_copy(
                noded_hbm.at[idxd_v.at[pl.ds(off, CH)]], bufd0, gd0)
            ds0 = pltpu.async_copy(
                nodes_hbm.at[idxs_v.at[pl.ds(off, CH)]], bufs0, gs0)
            dd1 = pltpu.async_copy(
                noded_hbm.at[idxd_v.at[pl.ds(off + CH, CH)]], bufd1, gd1)
            ds1 = pltpu.async_copy(
                nodes_hbm.at[idxs_v.at[pl.ds(off + CH, CH)]], bufs1, gs1)
            dd0.wait()
            w0 = pltpu.async_copy(
                bufd0.at[:, pl.ds(0, 128)],
                outm_hbm.at[pl.ds(base + off, CH)], wsem)
            w0b = pltpu.async_copy(
                bufd0.at[:, pl.ds(128, 128)],
                outg_hbm.at[pl.ds(base + off, CH)], wsem)
            ds0.wait()
            w1 = pltpu.async_copy(bufs0, outs_hbm.at[pl.ds(base + off, CH)],
                                  wsem)
            dd1.wait()
            w2 = pltpu.async_copy(
                bufd1.at[:, pl.ds(0, 128)],
                outm_hbm.at[pl.ds(base + off + CH, CH)], wsem)
            w2b = pltpu.async_copy(
                bufd1.at[:, pl.ds(128, 128)],
                outg_hbm.at[pl.ds(base + off + CH, CH)], wsem)
            ds1.wait()
            w3 = pltpu.async_copy(
                bufs1, outs_hbm.at[pl.ds(base + off + CH, CH)], wsem)
            w0.wait()
            w0b.wait()
            w1.wait()
            w2.wait()
            w2b.wait()
            w3.wait()
            return carry

        lax.fori_loop(0, nch // 2, pair, 0)

    return k(noded, nodes, dst_p, src_p)


def _edge_scalars(dist_2d, src_2d, dst_2d):
    rows = dist_2d.shape[0]

    def body(d_ref, s_ref, t_ref, env_ref, iss_ref):
        d = d_ref[...]
        env_ref[...] = 0.5 * (jnp.cos(jnp.pi * d / CUTOFF) + 1.0) * (
            d < CUTOFF).astype(jnp.float32)
        iss_ref[...] = (s_ref[...] == t_ref[...]).astype(jnp.float32)

    full = lambda shape: pl.BlockSpec(shape, lambda: tuple(0 for _ in shape))
    return pl.pallas_call(
        body,
        in_specs=[full((rows, CH))] * 3,
        out_specs=[full((rows, CH))] * 2,
        out_shape=[jax.ShapeDtypeStruct((rows, CH), jnp.float32)] * 2,
    )(dist_2d, src_2d, dst_2d)


def _edge_mlp(predm, predg, pres, dist2, env2, iss2,
              w1r, w1s, b1, w2, b2, w3, b3, gr, gsr, gb1, gw2r, gb2r, ep):
    eb = 1024
    grid = (ep // eb,)

    def body(predm_ref, predg_ref, pres_ref, d_ref, env_ref, iss_ref,
             w1r_ref, w1s_ref,
             b1_ref, w2_ref, b2_ref, w3_ref, b3_ref, gr_ref, gsr_ref, gb1_ref,
             gw2_ref, gb2_ref, msg_ref):
        d = d_ref[...]
        cen = lax.broadcasted_iota(jnp.int32, (eb, RBF_DIM), 1).astype(
            jnp.float32) * (CUTOFF / (RBF_DIM - 1))
        rbf = jnp.exp(_COEFF * (d - cen) ** 2)
        isself = iss_ref[...]
        pre_m = (predm_ref[...]
                 + jnp.dot(rbf, w1r_ref[...], preferred_element_type=jnp.float32)
                 + isself * w1s_ref[...] + b1_ref[...])
        m1 = _silu(pre_m)
        m2 = _silu(jnp.dot(m1, w2_ref[...], preferred_element_type=jnp.float32)
                   + b2_ref[...])
        mv = jnp.dot(m2, w3_ref[...], preferred_element_type=jnp.float32) \
            + b3_ref[...]
        pre_g = (pres_ref[...] + predg_ref[...]
                 + jnp.dot(rbf, gr_ref[...], preferred_element_type=jnp.float32)
                 + isself * gsr_ref[...] + gb1_ref[...])
        g1 = _silu(pre_g)
        logit = jnp.sum(g1 * gw2_ref[...], axis=1, keepdims=True) + gb2_ref[...]
        gate = _sigmoid(logit)
        msg_ref[...] = mv * (gate * env_ref[...])

    full = lambda shape: pl.BlockSpec(shape, lambda i: tuple(0 for _ in shape))
    return pl.pallas_call(
        body,
        grid=grid,
        in_specs=[
            pl.BlockSpec((eb, 128), lambda i: (i, 0)),
            pl.BlockSpec((eb, 128), lambda i: (i, 0)),
            pl.BlockSpec((eb, 128), lambda i: (i, 0)),
            pl.BlockSpec((eb, 1), lambda i: (i, 0)),
            pl.BlockSpec((eb, 1), lambda i: (i, 0)),
            pl.BlockSpec((eb, 1), lambda i: (i, 0)),
            full((16, 128)), full((1, 128)), full((1, 128)),
            full((128, 128)), full((1, 128)), full((128, 64)), full((1, 64)),
            full((16, 128)), full((1, 128)), full((1, 128)),
            full((1, 128)), full((1, 1)),
        ],
        out_specs=pl.BlockSpec((eb, 64), lambda i: (i, 0)),
        out_shape=jax.ShapeDtypeStruct((ep, 64), jnp.float32),
    )(predm, predg, pres, dist2, env2, iss2,
      w1r, w1s, b1, w2, b2, w3, b3, gr, gsr, gb1, gw2r, gb2r)


def _scatter_add(msg1, msg2, src2d, half, ep):
    nchunks = ep // CH
    cht = nchunks // NS
    acc_rows = half + CH
    stripe = acc_rows // NS
    mesh = plsc.VectorSubcoreMesh(core_axis_name="c", subcore_axis_name="s",
                                  num_cores=NC, num_subcores=NS)

    @functools.partial(
        pl.kernel,
        out_type=jax.ShapeDtypeStruct((NC, acc_rows, LATENT), jnp.float32),
        mesh=mesh,
        compiler_params=pltpu.CompilerParams(use_tc_tiling_on_sc=False),
        scratch_types=[
            pltpu.VMEM((cht, CH), jnp.int32),
            pltpu.VMEM((CH, LATENT), jnp.float32),
            pltpu.VMEM((stripe, LATENT), jnp.float32),
            pltpu.VMEM_SHARED((acc_rows, LATENT), jnp.float32),
            pltpu.SemaphoreType.DMA,
        ],
    )
    def k(msg1_hbm, msg2_hbm, src_hbm, part_hbm, idx_v, buf_v, obuf_v,
          acc_sh, sem):
        c = lax.axis_index("c")
        s = lax.axis_index("s")
        lo = c * half
        zv = jnp.zeros((16,), jnp.float32)

        def zrow(i, carry):
            for kk in range(LATENT // 16):
                obuf_v[i, pl.ds(kk * 16, 16)] = zv
            return carry

        lax.fori_loop(0, stripe, zrow, 0)
        pltpu.sync_copy(obuf_v, acc_sh.at[pl.ds(s * stripe, stripe)])
        pltpu.sync_copy(src_hbm.at[pl.ds(s * cht, cht)], idx_v)

        def remap(i, carry):
            for kk in range(CH // 16):
                g = idx_v[i, pl.ds(kk * 16, 16)]
                loc = g - lo
                ok = (loc >= 0) & (loc < half)
                idx_v[i, pl.ds(kk * 16, 16)] = jnp.where(ok, loc, half)
            return carry

        lax.fori_loop(0, cht, remap, 0)
        plsc.subcore_barrier()

        hs = NS // 2

        @pl.when(s < hs)
        def _():
            def chunk(j, carry):
                off = pl.multiple_of(s * cht * CH, CH) + j * CH
                pltpu.sync_copy(msg1_hbm.at[pl.ds(off, CH)], buf_v)
                pltpu.sync_copy(buf_v, acc_sh.at[idx_v.at[j]], add=True)
                return carry

            lax.fori_loop(0, cht, chunk, 0)

        @pl.when(s >= hs)
        def _():
            def chunk(j, carry):
                off = pl.multiple_of((s - hs) * cht * CH, CH) + j * CH
                pltpu.sync_copy(msg2_hbm.at[pl.ds(off, CH)], buf_v)
                pltpu.sync_copy(buf_v, acc_sh.at[idx_v.at[j]], add=True)
                return carry

            lax.fori_loop(0, cht, chunk, 0)
        plsc.subcore_barrier()
        pltpu.sync_copy(acc_sh.at[pl.ds(s * stripe, stripe)], obuf_v)
        pltpu.sync_copy(obuf_v, part_hbm.at[c, pl.ds(s * stripe, stripe)])

    return k(msg1, msg2, src2d)


def _output_mlp(part, efp, egw1, egb1, egw2, egb2, egw3, egb3,
                ow1, ob1, ow2, ob2, half, ne):
    nb = 640
    nbc = half // nb
    grid = (NC * nbc,)

    def body(p_ref, ef_ref, egw1_ref, egb1_ref, egw2_ref, egb2_ref,
             egw3_ref, egb3_ref, ow1_ref, ob1_ref, ow2_ref, ob2_ref, out_ref):
        p = p_ref[0]
        e1 = _silu(jnp.dot(ef_ref[...], egw1_ref[...],
                           preferred_element_type=jnp.float32) + egb1_ref[...])
        e2 = _silu(jnp.dot(e1, egw2_ref[...],
                           preferred_element_type=jnp.float32) + egb2_ref[...])
        egate = jnp.dot(e2, egw3_ref[...],
                        preferred_element_type=jnp.float32) + egb3_ref[...]
        for j in range(ne):
            x = p * egate[j:j + 1, :]
            o1 = _silu(jnp.dot(x, ow1_ref[...],
                               preferred_element_type=jnp.float32) + ob1_ref[...])
            o = jnp.dot(o1, ow2_ref[...],
                        preferred_element_type=jnp.float32) + ob2_ref[...]
            out_ref[:, j * LATENT:(j + 1) * LATENT] = o

    full = lambda shape: pl.BlockSpec(shape, lambda i: tuple(0 for _ in shape))
    return pl.pallas_call(
        body,
        grid=grid,
        in_specs=[
            pl.BlockSpec((1, nb, LATENT), lambda i: (i // nbc, i % nbc, 0)),
            full((8, 16)),
            full((16, 128)), full((1, 128)),
            full((128, 128)), full((1, 128)),
            full((128, 64)), full((1, 64)),
            full((64, 128)), full((1, 128)),
            full((128, 64)), full((1, 64)),
        ],
        out_specs=pl.BlockSpec((nb, ne * LATENT), lambda i: (i, 0)),
        out_shape=jax.ShapeDtypeStruct((NC * half, ne * LATENT), jnp.float32),
    )(part, efp, egw1, egb1, egw2, egb2, egw3, egb3, ow1, ob1, ow2, ob2)


def kernel(h, z, mask, e_feat, att_src, att_dst, att_dist, z_emb_W,
           msg_W1, msg_b1, msg_W2, msg_b2, msg_W3, msg_b3,
           gate_W1, gate_b1, gate_W2, gate_b2,
           eg_W1, eg_b1, eg_W2, eg_b2, eg_W3, eg_b3,
           out_W1, out_b1, out_W2, out_b2):
    bsz, n_atoms, h_dim = h.shape
    flat = bsz * n_atoms
    ne = e_feat.shape[0]
    e_att = att_src.shape[0]
    ep = ((e_att + NW * CH - 1) // (NW * CH)) * (NW * CH)
    pad = ep - e_att

    h_flat = h.reshape(flat, h_dim)
    z2 = z.reshape(flat, 1).astype(jnp.int32)
    zpad = jnp.pad(z_emb_W, ((0, 128 - z_emb_W.shape[0]), (0, 0)))

    w1h = msg_W1[0:128]
    w1z = msg_W1[128:160]
    w1s = msg_W1[160:161]
    w1r = msg_W1[161:177]
    gs = gate_W1[0:128]
    gd = gate_W1[128:256]
    gr = gate_W1[256:272]
    gsr = gate_W1[272:273]

    src_p = jnp.concatenate(
        [att_src.astype(jnp.int32), jnp.zeros((pad,), jnp.int32)])
    dst_p = jnp.concatenate(
        [att_dst.astype(jnp.int32), jnp.zeros((pad,), jnp.int32)])
    dist_p = jnp.concatenate(
        [att_dist.astype(jnp.float32), jnp.full((pad,), 2.0 * CUTOFF)])

    noded, nodes = _node_tables(h_flat, z2, zpad, w1h, w1z, gs, gd)
    env_2d, iss_2d = _edge_scalars(dist_p.reshape(ep // CH, CH),
                                   src_p.reshape(ep // CH, CH),
                                   dst_p.reshape(ep // CH, CH))
    eh = ep // 2
    msgs = []
    gathered = [
        _edge_gather(noded, nodes, dst_p[h * eh:(h + 1) * eh],
                     src_p[h * eh:(h + 1) * eh], eh)
        for h in range(2)
    ]
    for h in range(2):
        predm, predg, pres = gathered[h]
        msgs.append(_edge_mlp(
            predm, predg, pres, dist_p[h * eh:(h + 1) * eh].reshape(eh, 1),
            env_2d.reshape(ep, 1)[h * eh:(h + 1) * eh],
            iss_2d.reshape(ep, 1)[h * eh:(h + 1) * eh],
            w1r, w1s, msg_b1.reshape(1, -1), msg_W2, msg_b2.reshape(1, -1),
            msg_W3, msg_b3.reshape(1, -1), gr, gsr, gate_b1.reshape(1, -1),
            gate_W2.reshape(1, -1), gate_b2.reshape(1, 1), eh))
    src2d = src_p.reshape(ep // CH, CH)
    half = ((flat + 2 * 8 * NS - 1) // (2 * 8 * NS)) * (8 * NS)
    part = _scatter_add(msgs[0], msgs[1], src2d, half, ep)
    efp = jnp.pad(e_feat, ((0, 8 - ne), (0, 0)))
    out = _output_mlp(part, efp, eg_W1, eg_b1.reshape(1, -1), eg_W2,
                      eg_b2.reshape(1, -1), eg_W3, eg_b3.reshape(1, -1),
                      out_W1, out_b1.reshape(1, -1), out_W2,
                      out_b2.reshape(1, -1), half, ne)
    return out[:flat].reshape(bsz, n_atoms, ne, LATENT)


# speedup vs baseline: 1.5304x; 1.0227x over previous
import functools

import jax
import jax.numpy as jnp
from jax import lax
from jax.experimental import pallas as pl
from jax.experimental.pallas import tpu as pltpu
from jax.experimental.pallas import tpu_sc as plsc

CUTOFF = 5.0
RBF_DIM = 16
ATOM_DIM = 128
HIDDEN = 128
LATENT = 64

NC, NS = 2, 16
NW = NC * NS
CH = 128
_COEFF = -0.5 / (CUTOFF / (RBF_DIM - 1)) ** 2


def _silu(x):
    return x * (0.5 + 0.5 * jnp.tanh(0.5 * x))


def _sigmoid(x):
    return 0.5 + 0.5 * jnp.tanh(0.5 * x)


def _node_tables(h_flat, z2, zpad, w1h, w1z, gs, gd):
    flat = h_flat.shape[0]
    nb = 2000
    grid = (flat // nb,)

    def body(h_ref, z_ref, zp_ref, w1h_ref, w1z_ref, gs_ref, gd_ref,
             noded_ref, nodes_ref):
        h = h_ref[...]
        zc = z_ref[...]
        oh = (zc == lax.broadcasted_iota(jnp.int32, (nb, 128), 1)
              ).astype(jnp.float32)
        wzp = jnp.dot(zp_ref[...], w1z_ref[...],
                      preferred_element_type=jnp.float32)
        node_m = (jnp.dot(h, w1h_ref[...], preferred_element_type=jnp.float32)
                  + jnp.dot(oh, wzp, preferred_element_type=jnp.float32))
        node_gd = jnp.dot(h, gd_ref[...], preferred_element_type=jnp.float32)
        noded_ref[...] = jnp.concatenate([node_m, node_gd], axis=1)
        nodes_ref[...] = jnp.dot(h, gs_ref[...],
                                 preferred_element_type=jnp.float32)

    return pl.pallas_call(
        body,
        grid=grid,
        in_specs=[
            pl.BlockSpec((nb, 128), lambda i: (i, 0)),
            pl.BlockSpec((nb, 1), lambda i: (i, 0)),
            pl.BlockSpec((128, 32), lambda i: (0, 0)),
            pl.BlockSpec((128, 128), lambda i: (0, 0)),
            pl.BlockSpec((32, 128), lambda i: (0, 0)),
            pl.BlockSpec((128, 128), lambda i: (0, 0)),
            pl.BlockSpec((128, 128), lambda i: (0, 0)),
        ],
        out_specs=[
            pl.BlockSpec((nb, 256), lambda i: (i, 0)),
            pl.BlockSpec((nb, 128), lambda i: (i, 0)),
        ],
        out_shape=[
            jax.ShapeDtypeStruct((flat, 256), jnp.float32),
            jax.ShapeDtypeStruct((flat, 128), jnp.float32),
        ],
    )(h_flat, z2, zpad, w1h, w1z, gs, gd)


def _edge_gather(noded, nodes, dst_p, src_p, ep):
    ept = ep // NW
    nch = ept // CH
    mesh = plsc.VectorSubcoreMesh(core_axis_name="c", subcore_axis_name="s",
                                  num_cores=NC, num_subcores=NS)

    @functools.partial(
        pl.kernel,
        out_type=[
            jax.ShapeDtypeStruct((ep, 128), jnp.float32),
            jax.ShapeDtypeStruct((ep, 128), jnp.float32),
            jax.ShapeDtypeStruct((ep, 128), jnp.float32),
        ],
        mesh=mesh,
        compiler_params=pltpu.CompilerParams(use_tc_tiling_on_sc=False),
        scratch_types=[
            pltpu.VMEM((ept,), jnp.int32),
            pltpu.VMEM((ept,), jnp.int32),
            pltpu.VMEM((CH, 256), jnp.float32),
            pltpu.VMEM((CH, 256), jnp.float32),
            pltpu.VMEM((CH, 128), jnp.float32),
            pltpu.VMEM((CH, 128), jnp.float32),
            pltpu.SemaphoreType.DMA,
            pltpu.SemaphoreType.DMA,
            pltpu.SemaphoreType.DMA,
            pltpu.SemaphoreType.DMA,
            pltpu.SemaphoreType.DMA,
        ],
    )
    def k(noded_hbm, nodes_hbm, dst_hbm, src_hbm, outm_hbm, outg_hbm,
          outs_hbm, idxd_v, idxs_v, bufd0, bufd1, bufs0, bufs1,
          gd0, gd1, gs0, gs1, wsem):
        wid = lax.axis_index("s") * NC + lax.axis_index("c")
        base = pl.multiple_of(wid * ept, CH)
        pltpu.sync_copy(dst_hbm.at[pl.ds(base, ept)], idxd_v)
        pltpu.sync_copy(src_hbm.at[pl.ds(base, ept)], idxs_v)

        def pair(i, carry):
            off = pl.multiple_of(i * (2 * CH), CH)
            dd0 = pltpu.async_copy(
                noded_hbm.at[idxd_v.at[pl.ds(off, CH)]], bufd0, gd0)
            ds0 = pltpu.async_copy(
                nodes_hbm.at[idxs_v.at[pl.ds(off, CH)]], bufs0, gs0)
            dd1 = pltpu.async_copy(
                noded_hbm.at[idxd_v.at[pl.ds(off + CH, CH)]], bufd1, gd1)
            ds1 = pltpu.async_copy(
                nodes_hbm.at[idxs_v.at[pl.ds(off + CH, CH)]], bufs1, gs1)
            dd0.wait()
            w0 = pltpu.async_copy(
                bufd0.at[:, pl.ds(0, 128)],
                outm_hbm.at[pl.ds(base + off, CH)], wsem)
            w0b = pltpu.async_copy(
                bufd0.at[:, pl.ds(128, 128)],
                outg_hbm.at[pl.ds(base + off, CH)], wsem)
            ds0.wait()
            w1 = pltpu.async_copy(bufs0, outs_hbm.at[pl.ds(base + off, CH)],
                                  wsem)
            dd1.wait()
            w2 = pltpu.async_copy(
                bufd1.at[:, pl.ds(0, 128)],
                outm_hbm.at[pl.ds(base + off + CH, CH)], wsem)
            w2b = pltpu.async_copy(
                bufd1.at[:, pl.ds(128, 128)],
                outg_hbm.at[pl.ds(base + off + CH, CH)], wsem)
            ds1.wait()
            w3 = pltpu.async_copy(
                bufs1, outs_hbm.at[pl.ds(base + off + CH, CH)], wsem)
            w0.wait()
            w0b.wait()
            w1.wait()
            w2.wait()
            w2b.wait()
            w3.wait()
            return carry

        lax.fori_loop(0, nch // 2, pair, 0)

    return k(noded, nodes, dst_p, src_p)


def _edge_scalars(dist_2d, src_2d, dst_2d):
    rows = dist_2d.shape[0]

    def body(d_ref, s_ref, t_ref, env_ref, iss_ref):
        d = d_ref[...]
        env_ref[...] = 0.5 * (jnp.cos(jnp.pi * d / CUTOFF) + 1.0) * (
            d < CUTOFF).astype(jnp.float32)
        iss_ref[...] = (s_ref[...] == t_ref[...]).astype(jnp.float32)

    full = lambda shape: pl.BlockSpec(shape, lambda: tuple(0 for _ in shape))
    return pl.pallas_call(
        body,
        in_specs=[full((rows, CH))] * 3,
        out_specs=[full((rows, CH))] * 2,
        out_shape=[jax.ShapeDtypeStruct((rows, CH), jnp.float32)] * 2,
    )(dist_2d, src_2d, dst_2d)


def _edge_mlp(predm, predg, pres, dist2, env2, iss2,
              w1r, w1s, b1, w2, b2, w3, b3, gr, gsr, gb1, gw2r, gb2r, ep):
    eb = 1024
    grid = (ep // eb,)

    def body(predm_ref, predg_ref, pres_ref, d_ref, env_ref, iss_ref,
             w1r_ref, w1s_ref,
             b1_ref, w2_ref, b2_ref, w3_ref, b3_ref, gr_ref, gsr_ref, gb1_ref,
             gw2_ref, gb2_ref, msg_ref):
        d = d_ref[...]
        cen = lax.broadcasted_iota(jnp.int32, (eb, RBF_DIM), 1).astype(
            jnp.float32) * (CUTOFF / (RBF_DIM - 1))
        rbf = jnp.exp(_COEFF * (d - cen) ** 2)
        isself = iss_ref[...]
        pre_m = (predm_ref[...]
                 + jnp.dot(rbf, w1r_ref[...], preferred_element_type=jnp.float32)
                 + isself * w1s_ref[...] + b1_ref[...])
        m1 = _silu(pre_m)
        m2 = _silu(jnp.dot(m1, w2_ref[...], preferred_element_type=jnp.float32)
                   + b2_ref[...])
        mv = jnp.dot(m2, w3_ref[...], preferred_element_type=jnp.float32) \
            + b3_ref[...]
        pre_g = (pres_ref[...] + predg_ref[...]
                 + jnp.dot(rbf, gr_ref[...], preferred_element_type=jnp.float32)
                 + isself * gsr_ref[...] + gb1_ref[...])
        g1 = _silu(pre_g)
        logit = jnp.sum(g1 * gw2_ref[...], axis=1, keepdims=True) + gb2_ref[...]
        gate = _sigmoid(logit)
        msg_ref[...] = mv * (gate * env_ref[...])

    full = lambda shape: pl.BlockSpec(shape, lambda i: tuple(0 for _ in shape))
    return pl.pallas_call(
        body,
        grid=grid,
        in_specs=[
            pl.BlockSpec((eb, 128), lambda i: (i, 0)),
            pl.BlockSpec((eb, 128), lambda i: (i, 0)),
            pl.BlockSpec((eb, 128), lambda i: (i, 0)),
            pl.BlockSpec((eb, 1), lambda i: (i, 0)),
            pl.BlockSpec((eb, 1), lambda i: (i, 0)),
            pl.BlockSpec((eb, 1), lambda i: (i, 0)),
            full((16, 128)), full((1, 128)), full((1, 128)),
            full((128, 128)), full((1, 128)), full((128, 64)), full((1, 64)),
            full((16, 128)), full((1, 128)), full((1, 128)),
            full((1, 128)), full((1, 1)),
        ],
        out_specs=pl.BlockSpec((eb, 64), lambda i: (i, 0)),
        out_shape=jax.ShapeDtypeStruct((ep, 64), jnp.float32),
    )(predm, predg, pres, dist2, env2, iss2,
      w1r, w1s, b1, w2, b2, w3, b3, gr, gsr, gb1, gw2r, gb2r)


def _scatter_add(msgq, src2d, half, ep):
    nchunks = ep // CH
    cht = nchunks // NS
    acc_rows = half + CH
    stripe = acc_rows // NS
    mesh = plsc.VectorSubcoreMesh(core_axis_name="c", subcore_axis_name="s",
                                  num_cores=NC, num_subcores=NS)

    @functools.partial(
        pl.kernel,
        out_type=jax.ShapeDtypeStruct((NC, acc_rows, LATENT), jnp.float32),
        mesh=mesh,
        compiler_params=pltpu.CompilerParams(use_tc_tiling_on_sc=False),
        scratch_types=[
            pltpu.VMEM((cht, CH), jnp.int32),
            pltpu.VMEM((CH, LATENT), jnp.float32),
            pltpu.VMEM((stripe, LATENT), jnp.float32),
            pltpu.VMEM_SHARED((acc_rows, LATENT), jnp.float32),
            pltpu.SemaphoreType.DMA,
        ],
    )
    def k(m0_hbm, m1_hbm, m2_hbm, m3_hbm, src_hbm, part_hbm, idx_v, buf_v,
          obuf_v, acc_sh, sem):
        c = lax.axis_index("c")
        s = lax.axis_index("s")
        lo = c * half
        zv = jnp.zeros((16,), jnp.float32)

        def zrow(i, carry):
            for kk in range(LATENT // 16):
                obuf_v[i, pl.ds(kk * 16, 16)] = zv
            return carry

        lax.fori_loop(0, stripe, zrow, 0)
        pltpu.sync_copy(obuf_v, acc_sh.at[pl.ds(s * stripe, stripe)])
        pltpu.sync_copy(src_hbm.at[pl.ds(s * cht, cht)], idx_v)

        def remap(i, carry):
            for kk in range(CH // 16):
                g = idx_v[i, pl.ds(kk * 16, 16)]
                loc = g - lo
                ok = (loc >= 0) & (loc < half)
                idx_v[i, pl.ds(kk * 16, 16)] = jnp.where(ok, loc, half)
            return carry

        lax.fori_loop(0, cht, remap, 0)
        plsc.subcore_barrier()

        gs_ = NS // 4

        for qi, mq in enumerate((m0_hbm, m1_hbm, m2_hbm, m3_hbm)):
            @pl.when((s >= qi * gs_) & (s < (qi + 1) * gs_))
            def _(mq=mq, qi=qi):
                def chunk(j, carry):
                    off = pl.multiple_of((s - qi * gs_) * cht * CH, CH) \
                        + j * CH
                    pltpu.sync_copy(mq.at[pl.ds(off, CH)], buf_v)
                    pltpu.sync_copy(buf_v, acc_sh.at[idx_v.at[j]], add=True)
                    return carry

                lax.fori_loop(0, cht, chunk, 0)
        plsc.subcore_barrier()
        pltpu.sync_copy(acc_sh.at[pl.ds(s * stripe, stripe)], obuf_v)
        pltpu.sync_copy(obuf_v, part_hbm.at[c, pl.ds(s * stripe, stripe)])

    return k(*msgq, src2d)


def _output_mlp(part, efp, egw1, egb1, egw2, egb2, egw3, egb3,
                ow1, ob1, ow2, ob2, half, ne):
    nb = 640
    nbc = half // nb
    grid = (NC * nbc,)

    def body(p_ref, ef_ref, egw1_ref, egb1_ref, egw2_ref, egb2_ref,
             egw3_ref, egb3_ref, ow1_ref, ob1_ref, ow2_ref, ob2_ref, out_ref):
        p = p_ref[0]
        e1 = _silu(jnp.dot(ef_ref[...], egw1_ref[...],
                           preferred_element_type=jnp.float32) + egb1_ref[...])
        e2 = _silu(jnp.dot(e1, egw2_ref[...],
                           preferred_element_type=jnp.float32) + egb2_ref[...])
        egate = jnp.dot(e2, egw3_ref[...],
                        preferred_element_type=jnp.float32) + egb3_ref[...]
        for j in range(ne):
            x = p * egate[j:j + 1, :]
            o1 = _silu(jnp.dot(x, ow1_ref[...],
                               preferred_element_type=jnp.float32) + ob1_ref[...])
            o = jnp.dot(o1, ow2_ref[...],
                        preferred_element_type=jnp.float32) + ob2_ref[...]
            out_ref[:, j * LATENT:(j + 1) * LATENT] = o

    full = lambda shape: pl.BlockSpec(shape, lambda i: tuple(0 for _ in shape))
    return pl.pallas_call(
        body,
        grid=grid,
        in_specs=[
            pl.BlockSpec((1, nb, LATENT), lambda i: (i // nbc, i % nbc, 0)),
            full((8, 16)),
            full((16, 128)), full((1, 128)),
            full((128, 128)), full((1, 128)),
            full((128, 64)), full((1, 64)),
            full((64, 128)), full((1, 128)),
            full((128, 64)), full((1, 64)),
        ],
        out_specs=pl.BlockSpec((nb, ne * LATENT), lambda i: (i, 0)),
        out_shape=jax.ShapeDtypeStruct((NC * half, ne * LATENT), jnp.float32),
    )(part, efp, egw1, egb1, egw2, egb2, egw3, egb3, ow1, ob1, ow2, ob2)


def kernel(h, z, mask, e_feat, att_src, att_dst, att_dist, z_emb_W,
           msg_W1, msg_b1, msg_W2, msg_b2, msg_W3, msg_b3,
           gate_W1, gate_b1, gate_W2, gate_b2,
           eg_W1, eg_b1, eg_W2, eg_b2, eg_W3, eg_b3,
           out_W1, out_b1, out_W2, out_b2):
    bsz, n_atoms, h_dim = h.shape
    flat = bsz * n_atoms
    ne = e_feat.shape[0]
    e_att = att_src.shape[0]
    ep = ((e_att + NW * CH - 1) // (NW * CH)) * (NW * CH)
    pad = ep - e_att

    h_flat = h.reshape(flat, h_dim)
    z2 = z.reshape(flat, 1).astype(jnp.int32)
    zpad = jnp.pad(z_emb_W, ((0, 128 - z_emb_W.shape[0]), (0, 0)))

    w1h = msg_W1[0:128]
    w1z = msg_W1[128:160]
    w1s = msg_W1[160:161]
    w1r = msg_W1[161:177]
    gs = gate_W1[0:128]
    gd = gate_W1[128:256]
    gr = gate_W1[256:272]
    gsr = gate_W1[272:273]

    src_p = jnp.concatenate(
        [att_src.astype(jnp.int32), jnp.zeros((pad,), jnp.int32)])
    dst_p = jnp.concatenate(
        [att_dst.astype(jnp.int32), jnp.zeros((pad,), jnp.int32)])
    dist_p = jnp.concatenate(
        [att_dist.astype(jnp.float32), jnp.full((pad,), 2.0 * CUTOFF)])

    noded, nodes = _node_tables(h_flat, z2, zpad, w1h, w1z, gs, gd)
    env_2d, iss_2d = _edge_scalars(dist_p.reshape(ep // CH, CH),
                                   src_p.reshape(ep // CH, CH),
                                   dst_p.reshape(ep // CH, CH))
    eh = ep // 4
    msgs = []
    gathered = [
        _edge_gather(noded, nodes, dst_p[h * eh:(h + 1) * eh],
                     src_p[h * eh:(h + 1) * eh], eh)
        for h in range(4)
    ]
    for h in range(4):
        predm, predg, pres = gathered[h]
        msgs.append(_edge_mlp(
            predm, predg, pres, dist_p[h * eh:(h + 1) * eh].reshape(eh, 1),
            env_2d.reshape(ep, 1)[h * eh:(h + 1) * eh],
            iss_2d.reshape(ep, 1)[h * eh:(h + 1) * eh],
            w1r, w1s, msg_b1.reshape(1, -1), msg_W2, msg_b2.reshape(1, -1),
            msg_W3, msg_b3.reshape(1, -1), gr, gsr, gate_b1.reshape(1, -1),
            gate_W2.reshape(1, -1), gate_b2.reshape(1, 1), eh))
    src2d = src_p.reshape(ep // CH, CH)
    half = ((flat + 2 * 8 * NS - 1) // (2 * 8 * NS)) * (8 * NS)
    part = _scatter_add(msgs, src2d, half, ep)
    efp = jnp.pad(e_feat, ((0, 8 - ne), (0, 0)))
    out = _output_mlp(part, efp, eg_W1, eg_b1.reshape(1, -1), eg_W2,
                      eg_b2.reshape(1, -1), eg_W3, eg_b3.reshape(1, -1),
                      out_W1, out_b1.reshape(1, -1), out_W2,
                      out_b2.reshape(1, -1), half, ne)
    return out[:flat].reshape(bsz, n_atoms, ne, LATENT)
